# baseline (speedup 1.0000x reference)
import functools

import jax
import jax.numpy as jnp
from jax import lax
from jax.experimental import pallas as pl
from jax.experimental.pallas import tpu as pltpu
from jax.experimental.pallas import tpu_sc as plsc

D = 64
DP = 128
NC = 2
NS = 16
NW = NC * NS
L = 16
TR = 8
CAP = 704
CR = 256
IB = 1024


def _rsqrt_nr(x):
    bits = lax.bitcast_convert_type(x, jnp.int32)
    y = lax.bitcast_convert_type(
        jnp.int32(0x5F3759DF) - lax.shift_right_logical(bits, 1), jnp.float32
    )
    half = x * 0.5
    for _ in range(3):
        y = y * (1.5 - half * y * y)
    return y


def _make_kernel(batch, n_rows):
    n_tiles = n_rows // TR
    tiles_base = n_tiles // NW
    tiles_rem = n_tiles % NW
    n_dump = batch + ((NW + TR - 1) // TR) * TR
    mesh = plsc.VectorSubcoreMesh(core_axis_name="c", subcore_axis_name="s")

    @functools.partial(
        pl.kernel,
        mesh=mesh,
        out_type=jax.ShapeDtypeStruct((n_dump, DP), jnp.float32),
        compiler_params=pltpu.CompilerParams(needs_layout_passes=False),
        scratch_types=[
            pltpu.VMEM((IB,), jnp.int32),
            pltpu.VMEM((CAP,), jnp.int32),
            pltpu.VMEM((CAP,), jnp.int32),
            pltpu.VMEM((CR, D), jnp.float32),
            pltpu.VMEM((CAP, DP), jnp.float32),
            pltpu.SemaphoreType.DMA,
        ],
    )
    def k(table_hbm, idx_hbm, stage_hbm, ids_v, myrow_v, mypos_v, chunk_v,
          obuf_v, sem):
        wid = lax.axis_index("s") * NC + lax.axis_index("c")
        tile_lo = wid * tiles_base + jnp.minimum(wid, tiles_rem)
        my_tiles = tiles_base + (wid < tiles_rem).astype(jnp.int32)
        row_lo = tile_lo * TR
        row_cnt = my_tiles * TR
        row_hi = row_lo + row_cnt
        dump = batch + wid
        lanes = lax.iota(jnp.int32, L)

        def init_body(i, carry):
            mypos_v[pl.ds(i * L, L)] = jnp.broadcast_to(dump, (L,))
            return carry

        lax.fori_loop(0, CAP // L, init_body, 0)

        def filter_body(blk, cnt):
            pltpu.sync_copy(idx_hbm.at[pl.ds(blk * IB, IB)], ids_v)
            for v in range(IB // L):
                ids16 = ids_v[pl.ds(v * L, L)]
                m = (ids16 >= row_lo) & (ids16 < row_hi)
                c = plsc.all_reduce_population_count(m)[0]
                at = jnp.minimum(cnt, CAP - L)
                plsc.store_compressed(myrow_v.at[pl.ds(at, L)], ids16, mask=m)
                plsc.store_compressed(
                    mypos_v.at[pl.ds(at, L)],
                    lanes + (blk * IB + v * L),
                    mask=m,
                )
                cnt = cnt + c
            return cnt

        lax.fori_loop(0, batch // IB, filter_body, jnp.int32(0))

        n_chunks = (row_cnt + CR - 1) // CR

        def chunk_body(cidx, carry):
            clo = row_lo + jnp.minimum(cidx * CR, row_cnt - CR)
            chi = clo + CR
            pltpu.sync_copy(table_hbm.at[pl.ds(clo, CR)], chunk_v)

            def vreg_body(v, carry2):
                rows16 = myrow_v[pl.ds(v * L, L)]
                m = (rows16 >= clo) & (rows16 < chi)
                mi = m.astype(jnp.int32)
                have = plsc.all_reduce_population_count(m)[0]

                @pl.when(have > 0)
                def _process():
                    for j in range(L):
                        @pl.when(mi[j] != 0)
                        def _one():
                            off = rows16[j] - clo
                            vs = [
                                chunk_v[off, pl.ds(c * L, L)]
                                for c in range(D // L)
                            ]
                            s = vs[0] * vs[0]
                            for vv in vs[1:]:
                                s = s + vv * vv
                            sc = jnp.broadcast_to(
                                _rsqrt_nr(jnp.sum(s)), (L,)
                            )
                            for c, vv in enumerate(vs):
                                obuf_v[v * L + j, pl.ds(c * L, L)] = vv * sc
                return carry2

            lax.fori_loop(0, CAP // L, vreg_body, 0)
            return carry

        lax.fori_loop(0, n_chunks, chunk_body, 0)

        pltpu.async_copy(obuf_v, stage_hbm.at[mypos_v], sem).wait()

    return k


def kernel(user_ids, table):
    batch = user_ids.shape[0]
    k = _make_kernel(batch, table.shape[0])
    stage = k(table, user_ids)
    return stage[:batch, :D]


# speedup vs baseline: 1.4997x; 1.4997x over previous
import functools

import jax
import jax.numpy as jnp
from jax import lax
from jax.experimental import pallas as pl
from jax.experimental.pallas import tpu as pltpu
from jax.experimental.pallas import tpu_sc as plsc

D = 64
DP = 128
NC = 2
NS = 16
NW = NC * NS
L = 16
TR = 8
CAP = 704
NSUB = 16
CAPB = 64
CR = 128
IB = 1024
WCAP = 64


def _rsqrt_nr(x):
    bits = lax.bitcast_convert_type(x, jnp.int32)
    y = lax.bitcast_convert_type(
        jnp.int32(0x5F3759DF) - lax.shift_right_logical(bits, 1), jnp.float32
    )
    half = x * 0.5
    for _ in range(3):
        y = y * (1.5 - half * y * y)
    return y


def _make_kernel(batch, n_rows):
    n_tiles = n_rows // TR
    tiles_base = n_tiles // NW
    tiles_rem = n_tiles % NW
    n_dump = batch + ((NW + TR - 1) // TR) * TR
    mesh = plsc.VectorSubcoreMesh(core_axis_name="c", subcore_axis_name="s")

    @functools.partial(
        pl.kernel,
        mesh=mesh,
        out_type=jax.ShapeDtypeStruct((n_dump, DP), jnp.float32),
        compiler_params=pltpu.CompilerParams(needs_layout_passes=False),
        scratch_types=[
            pltpu.VMEM((IB,), jnp.int32),
            pltpu.VMEM((CAP,), jnp.int32),
            pltpu.VMEM((CAP,), jnp.int32),
            pltpu.VMEM(((NSUB + 1) * CAPB,), jnp.int32),
            pltpu.VMEM(((NSUB + 1) * CAPB,), jnp.int32),
            pltpu.VMEM((WCAP,), jnp.int32),
            pltpu.VMEM((WCAP,), jnp.int32),
            pltpu.VMEM((CR, D), jnp.float32),
            pltpu.VMEM((CR, D), jnp.float32),
            pltpu.VMEM((CAP, DP), jnp.float32),
            pltpu.SemaphoreType.DMA,
            pltpu.SemaphoreType.DMA,
        ],
    )
    def k(table_hbm, idx_hbm, stage_hbm, ids_v, myrow_v, mypos_v, brow_v,
          bslot_v, wrow_v, wslot_v, cha_v, chb_v, obuf_v, sem, sem2):
        wid = lax.axis_index("s") * NC + lax.axis_index("c")
        tile_lo = wid * tiles_base + jnp.minimum(wid, tiles_rem)
        my_tiles = tiles_base + (wid < tiles_rem).astype(jnp.int32)
        row_lo = tile_lo * TR
        row_cnt = my_tiles * TR
        row_hi = row_lo + row_cnt
        dump = batch + wid
        lanes = lax.iota(jnp.int32, L)
        neg1 = jnp.broadcast_to(jnp.int32(-1), (L,))

        def init_body(i, carry):
            mypos_v[pl.ds(i * L, L)] = jnp.broadcast_to(dump, (L,))
            return carry

        lax.fori_loop(0, CAP // L, init_body, 0)

        def binit_body(i, carry):
            brow_v[pl.ds(i * L, L)] = neg1
            return carry

        lax.fori_loop(0, (NSUB + 1) * CAPB // L, binit_body, 0)

        def filter_body(blk, cnt):
            pltpu.sync_copy(idx_hbm.at[pl.ds(blk * IB, IB)], ids_v)
            for v in range(IB // L):
                ids16 = ids_v[pl.ds(v * L, L)]
                m = (ids16 >= row_lo) & (ids16 < row_hi)
                c = plsc.all_reduce_population_count(m)[0]
                at = jnp.minimum(cnt, CAP - L)
                plsc.store_compressed(myrow_v.at[pl.ds(at, L)], ids16, mask=m)
                plsc.store_compressed(
                    mypos_v.at[pl.ds(at, L)],
                    lanes + (blk * IB + v * L),
                    mask=m,
                )
                cnt = cnt + c
            return cnt

        cnt = lax.fori_loop(0, batch // IB, filter_body, jnp.int32(0))
        cnt = jnp.minimum(cnt, CAP)

        def redist_body(v, bcnts):
            rows16 = myrow_v[pl.ds(v * L, L)]
            gidx = lanes + v * L
            valid = gidx < cnt
            sv = ((rows16 - row_lo) * NSUB) // row_cnt
            new = []
            for s in range(NSUB):
                m = valid & (sv == s)
                c = plsc.all_reduce_population_count(m)[0]
                at = s * CAPB + jnp.minimum(bcnts[s], CAPB - L)
                plsc.store_compressed(brow_v.at[pl.ds(at, L)], rows16, mask=m)
                plsc.store_compressed(bslot_v.at[pl.ds(at, L)], gidx, mask=m)
                new.append(bcnts[s] + c)
            return tuple(new)

        lax.fori_loop(0, CAP // L, redist_body,
                      tuple(jnp.int32(0) for _ in range(NSUB)))

        n_chunks = (row_cnt + CR - 1) // CR

        def chunk_lo(c):
            return row_lo + jnp.minimum(c * CR, row_cnt - CR)

        pltpu.async_copy(table_hbm.at[pl.ds(chunk_lo(0), CR)], cha_v, sem)

        def process(buf_v, clo, chi):
            base = (((clo - row_lo) * NSUB) // row_cnt) * CAPB
            wcnt = jnp.int32(0)
            for v in range(2 * CAPB // L):
                rows16 = brow_v[pl.ds(base + v * L, L)]
                slots16 = bslot_v[pl.ds(base + v * L, L)]
                m = (rows16 >= clo) & (rows16 < chi)
                c = plsc.all_reduce_population_count(m)[0]
                at = jnp.minimum(wcnt, WCAP - L)
                plsc.store_compressed(wrow_v.at[pl.ds(at, L)], rows16, mask=m)
                plsc.store_compressed(wslot_v.at[pl.ds(at, L)], slots16,
                                      mask=m)
                wcnt = wcnt + c

            def extract_body(g, carry):
                rows16 = wrow_v[pl.ds(g * L, L)]
                slots16 = wslot_v[pl.ds(g * L, L)]
                live = ((lanes + g * L) < wcnt).astype(jnp.int32)
                for j in range(L):
                    @pl.when(live[j] != 0)
                    def _one():
                        off = rows16[j] - clo
                        slot = slots16[j]
                        vs = [
                            buf_v[off, pl.ds(cc * L, L)]
                            for cc in range(D // L)
                        ]
                        s = vs[0] * vs[0]
                        for vv in vs[1:]:
                            s = s + vv * vv
                        sc = jnp.broadcast_to(_rsqrt_nr(jnp.sum(s)), (L,))
                        for cc, vv in enumerate(vs):
                            obuf_v[slot, pl.ds(cc * L, L)] = vv * sc
                return carry

            lax.fori_loop(0, (wcnt + L - 1) // L, extract_body, 0)

        def chunk_body(c, carry):
            clo = chunk_lo(c)
            chi = clo + CR
            pltpu.make_async_copy(
                table_hbm.at[pl.ds(0, CR)], cha_v, sem
            ).wait()
            nxt = chunk_lo(c + 1)
            par = c & 1

            @pl.when((c + 1 < n_chunks) & (par == 0))
            def _fire_b():
                pltpu.async_copy(table_hbm.at[pl.ds(nxt, CR)], chb_v, sem)

            @pl.when((c + 1 < n_chunks) & (par == 1))
            def _fire_a():
                pltpu.async_copy(table_hbm.at[pl.ds(nxt, CR)], cha_v, sem)

            @pl.when(par == 0)
            def _proc_a():
                process(cha_v, clo, chi)

            @pl.when(par == 1)
            def _proc_b():
                process(chb_v, clo, chi)

            return carry

        lax.fori_loop(0, n_chunks, chunk_body, 0)

        pltpu.async_copy(obuf_v, stage_hbm.at[mypos_v], sem2).wait()

    return k


def kernel(user_ids, table):
    batch = user_ids.shape[0]
    k = _make_kernel(batch, table.shape[0])
    stage = k(table, user_ids)
    return stage[:batch, :D]


# speedup vs baseline: 2.9126x; 1.9420x over previous
import functools

import jax
import jax.numpy as jnp
from jax import lax
from jax.experimental import pallas as pl
from jax.experimental.pallas import tpu as pltpu
from jax.experimental.pallas import tpu_sc as plsc

D = 64
NC = 2
NS = 16
NW = NC * NS
L = 16
NQ = 4


def _rsqrt_nr(x):
    bits = lax.bitcast_convert_type(x, jnp.int32)
    y = lax.bitcast_convert_type(
        jnp.int32(0x5F3759DF) - lax.shift_right_logical(bits, 1), jnp.float32
    )
    half = x * 0.5
    for _ in range(3):
        y = y * (1.5 - half * y * y)
    return y


def _make_kernel(batch):
    b_per_w = batch // NW
    mesh = plsc.VectorSubcoreMesh(core_axis_name="c", subcore_axis_name="s")

    @functools.partial(
        pl.kernel,
        mesh=mesh,
        out_type=jax.ShapeDtypeStruct((batch, D), jnp.float32),
        compiler_params=pltpu.CompilerParams(needs_layout_passes=False),
        scratch_types=[
            pltpu.VMEM((b_per_w,), jnp.int32),
            pltpu.VMEM((b_per_w, D), jnp.float32),
        ] + [pltpu.SemaphoreType.DMA] * NQ,
    )
    def k(table_hbm, idx_hbm, out_hbm, idx_v, rows_v, *sems):
        wid = lax.axis_index("s") * NC + lax.axis_index("c")
        base = wid * b_per_w
        pltpu.sync_copy(idx_hbm.at[pl.ds(base, b_per_w)], idx_v)

        def fire_body(g, carry):
            ids = idx_v[pl.ds(g * L, L)]
            for j in range(L):
                r = ids[j]
                pltpu.async_copy(
                    table_hbm.at[pl.ds(r, 1)],
                    rows_v.at[pl.ds(g * L + j, 1)],
                    sems[j % NQ],
                )
            return carry

        lax.fori_loop(0, b_per_w // L, fire_body, 0)

        def drain_body(i, carry):
            for q in range(NQ):
                pltpu.make_async_copy(
                    table_hbm.at[pl.ds(0, 1)], rows_v.at[pl.ds(0, 1)], sems[q]
                ).wait()
            return carry

        lax.fori_loop(0, b_per_w // NQ, drain_body, 0)

        def row_body(i, carry):
            vs = [rows_v[i, pl.ds(c * L, L)] for c in range(D // L)]
            s = vs[0] * vs[0]
            for v in vs[1:]:
                s = s + v * v
            sc = jnp.broadcast_to(_rsqrt_nr(jnp.sum(s)), (L,))
            for c, v in enumerate(vs):
                rows_v[i, pl.ds(c * L, L)] = v * sc
            return carry

        lax.fori_loop(0, b_per_w, row_body, 0, unroll=2)

        pltpu.sync_copy(rows_v, out_hbm.at[pl.ds(base, b_per_w)])

    return k


def kernel(user_ids, table):
    batch = user_ids.shape[0]
    k = _make_kernel(batch)
    return k(table, user_ids)
